# Initial kernel scaffold; baseline (speedup 1.0000x reference)
import jax
import jax.numpy as jnp
from jax.experimental import pallas as pl


def kernel(chn_llr, edge_var, edge_chk, net_w1, net_b1, net_w2, net_b2):
    raise NotImplementedError("write your pallas kernel here")


# speedup vs baseline: 5.2497x; 5.2497x over previous
import functools
import jax
import jax.numpy as jnp
from jax import lax
from jax.experimental import pallas as pl
from jax.experimental.pallas import tpu as pltpu
from jax.experimental.pallas import tpu_sc as plsc

N = 50000
M = 25000
E = 200000
B = 32
T = 10
LLR_CLIP = 15.0
ABS_MIN = 1.1920930376163597e-06
RATE = 0.5

NC = 2
NS = 16
L = 16

E_PAD = 200704
CH_PER_TILE = 98
CH_PER_SLAB = 14
SLABS = 7
SLAB_ROWS = CH_PER_SLAB * 128
N_SP = 50048
M_SP = 25088
NROWS_TILE = N_SP // NS
MROWS_TILE = M_SP // NS

_mesh = plsc.VectorSubcoreMesh(core_axis_name="c", subcore_axis_name="s",
                               num_cores=NC, num_subcores=NS)


def _zero_rows(zeros_hbm, acc, base, nrows):
    full = nrows // 128
    tail = nrows - full * 128

    def body(k, _):
        pltpu.sync_copy(zeros_hbm, acc.at[pl.ds(base + k * 128, 128)])
        return 0

    lax.fori_loop(0, full, body, 0)
    if tail:
        pltpu.sync_copy(zeros_hbm.at[pl.ds(0, tail)],
                        acc.at[pl.ds(base + full * 128, tail)])


def _preload_acc(wiell, acc, c, s):
    r0 = s * NROWS_TILE
    full = NROWS_TILE // 128
    tail = NROWS_TILE - full * 128

    def body(k, _):
        pltpu.sync_copy(wiell.at[c, pl.ds(r0 + k * 128, 128)],
                        acc.at[pl.ds(r0 + k * 128, 128)])
        return 0

    lax.fori_loop(0, full, body, 0)
    if tail:
        pltpu.sync_copy(wiell.at[c, pl.ds(r0 + full * 128, tail)],
                        acc.at[pl.ds(r0 + full * 128, tail)])


def _gather_from_acc(acc, gath, idxbuf, gbuf, sem, c, tbase):
    def outer(slab, _):
        r0g = tbase + slab * SLAB_ROWS

        def chunk(j, _):
            pltpu.async_copy(acc.at[idxbuf.at[slab * CH_PER_SLAB + j]],
                             gbuf.at[pl.ds(j * 128, 128)], sem).wait()
            return 0

        lax.fori_loop(0, CH_PER_SLAB, chunk, 0)
        pltpu.async_copy(gbuf, gath.at[c, pl.ds(r0g, SLAB_ROWS)], sem).wait()
        return 0

    lax.fori_loop(0, SLABS, outer, 0)


def _scatter_add_to_acc(vals, acc, idxbuf, vbuf, sem, c, tbase):
    def outer(slab, _):
        r0 = tbase + slab * SLAB_ROWS
        pltpu.async_copy(vals.at[c, pl.ds(r0, SLAB_ROWS)], vbuf, sem).wait()

        def chunk(j, _):
            pltpu.sync_copy(vbuf.at[pl.ds(j * 128, 128)],
                            acc.at[idxbuf.at[slab * CH_PER_SLAB + j]],
                            add=True)
            return 0

        lax.fori_loop(0, CH_PER_SLAB, chunk, 0)
        return 0

    lax.fori_loop(0, SLABS, outer, 0)


@functools.partial(
    pl.kernel,
    out_type=[jax.ShapeDtypeStruct((NC, N, L), jnp.float32),
              jax.ShapeDtypeStruct((NC, E_PAD, L), jnp.float32)],
    mesh=_mesh,
    scratch_types=[pltpu.VMEM((CH_PER_TILE, 128), jnp.int32),
                   pltpu.VMEM((SLAB_ROWS, L), jnp.float32),
                   pltpu.VMEM((SLAB_ROWS, L), jnp.float32),
                   pltpu.VMEM_SHARED((N_SP, L), jnp.float32),
                   pltpu.SemaphoreType.DMA],
    compiler_params=pltpu.CompilerParams(use_tc_tiling_on_sc=False),
)
def _sc_var(vals, idx3d, wiell, sums, gath, idxbuf, vbuf, gbuf, acc, sem):
    c = lax.axis_index("c")
    s = lax.axis_index("s")
    tbase = s * (CH_PER_TILE * 128)

    pltpu.sync_copy(idx3d.at[s], idxbuf)
    _preload_acc(wiell, acc, c, s)
    plsc.subcore_barrier()

    _scatter_add_to_acc(vals, acc, idxbuf, vbuf, sem, c, tbase)
    plsc.subcore_barrier()

    r0 = s * NROWS_TILE
    nfull = 24

    def sums_out(k, _):
        pltpu.sync_copy(acc.at[pl.ds(r0 + k * 128, 128)],
                        sums.at[c, pl.ds(r0 + k * 128, 128)])
        return 0

    lax.fori_loop(0, nfull, sums_out, 0)
    t0 = jnp.minimum(r0 + NROWS_TILE, N) - 128
    pltpu.sync_copy(acc.at[pl.ds(t0, 128)], sums.at[c, pl.ds(t0, 128)])

    _gather_from_acc(acc, gath, idxbuf, gbuf, sem, c, tbase)


@functools.partial(
    pl.kernel,
    out_type=jax.ShapeDtypeStruct((NC, E_PAD, L), jnp.float32),
    mesh=_mesh,
    scratch_types=[pltpu.VMEM((CH_PER_TILE, 128), jnp.int32),
                   pltpu.VMEM((SLAB_ROWS, L), jnp.float32),
                   pltpu.VMEM_SHARED((N_SP, L), jnp.float32),
                   pltpu.SemaphoreType.DMA],
    compiler_params=pltpu.CompilerParams(use_tc_tiling_on_sc=False),
)
def _sc_gather_init(idx3d, wiell, gath, idxbuf, gbuf, acc, sem):
    c = lax.axis_index("c")
    s = lax.axis_index("s")
    tbase = s * (CH_PER_TILE * 128)

    pltpu.sync_copy(idx3d.at[s], idxbuf)
    _preload_acc(wiell, acc, c, s)
    plsc.subcore_barrier()
    _gather_from_acc(acc, gath, idxbuf, gbuf, sem, c, tbase)


@functools.partial(
    pl.kernel,
    out_type=[jax.ShapeDtypeStruct((NC, E_PAD, L), jnp.float32),
              jax.ShapeDtypeStruct((NC, E_PAD, L), jnp.float32)],
    mesh=_mesh,
    scratch_types=[pltpu.VMEM((CH_PER_TILE, 128), jnp.int32),
                   pltpu.VMEM((SLAB_ROWS, L), jnp.float32),
                   pltpu.VMEM((SLAB_ROWS, L), jnp.float32),
                   pltpu.VMEM_SHARED((M_SP, L), jnp.float32),
                   pltpu.VMEM_SHARED((M_SP, L), jnp.float32),
                   pltpu.SemaphoreType.DMA],
    compiler_params=pltpu.CompilerParams(use_tc_tiling_on_sc=False),
)
def _sc_chk(valsA, valsB, idx3d, zeros_hbm, gathA, gathB,
            idxbuf, vbuf, gbuf, accA, accB, sem):
    c = lax.axis_index("c")
    s = lax.axis_index("s")
    tbase = s * (CH_PER_TILE * 128)

    pltpu.sync_copy(idx3d.at[s], idxbuf)
    _zero_rows(zeros_hbm, accA, s * MROWS_TILE, MROWS_TILE)
    _zero_rows(zeros_hbm, accB, s * MROWS_TILE, MROWS_TILE)
    plsc.subcore_barrier()

    _scatter_add_to_acc(valsA, accA, idxbuf, vbuf, sem, c, tbase)
    _scatter_add_to_acc(valsB, accB, idxbuf, vbuf, sem, c, tbase)
    plsc.subcore_barrier()

    _gather_from_acc(accA, gathA, idxbuf, gbuf, sem, c, tbase)
    _gather_from_acc(accB, gathB, idxbuf, gbuf, sem, c, tbase)


RV = E_PAD * L // 128
VBLK = 512
VGRID = RV // VBLK


def _edge_spec():
    return pl.BlockSpec((1, VBLK, 128), lambda c, i: (c, i, 0))


def _scal_spec():
    return pl.BlockSpec((1, 1, 128), lambda c, i: (c, 0, 0))


def _tc_v_body(mv_ref, mc_ref, gv_ref, g_ref, we_ref,
               mvo_ref, neg_ref, alog_ref):
    g = g_ref[...]
    we = we_ref[...]
    mv = (1.0 - g) * mv_ref[...] + g * (gv_ref[...] - we * mc_ref[...])
    mvo_ref[...] = mv
    lam = jnp.clip(mv, -LLR_CLIP, LLR_CLIP)
    neg_ref[...] = jnp.where(lam < 0, 1.0, 0.0).astype(jnp.float32)
    ab = jnp.clip(jnp.abs(lam), ABS_MIN, LLR_CLIP)
    alog_ref[...] = jnp.log(jnp.tanh(ab * 0.5))


def _tc_h_body(mc_ref, neg_ref, alog_ref, gneg_ref, glog_ref, g_ref, we_ref,
               mco_ref, a2_ref):
    g = g_ref[...]
    we = we_ref[...]
    neg = neg_ref[...]
    parity = jnp.mod(gneg_ref[...] - neg, 2.0)
    sgn = 1.0 - 2.0 * parity
    amp = glog_ref[...] - alog_ref[...]
    t = jnp.exp(amp) * (1.0 - 1e-6)
    h = sgn * jnp.log((1.0 + t) / (1.0 - t))
    mc = (1.0 - g) * mc_ref[...] + g * h
    mco_ref[...] = mc
    a2_ref[...] = we * mc


def _tc_v(mv, mc, gv, gam, we):
    return pl.pallas_call(
        _tc_v_body,
        grid=(NC, VGRID),
        in_specs=[_edge_spec(), _edge_spec(), _edge_spec(),
                  _scal_spec(), _scal_spec()],
        out_specs=[_edge_spec(), _edge_spec(), _edge_spec()],
        out_shape=[jax.ShapeDtypeStruct((NC, RV, 128), jnp.float32)] * 3,
        compiler_params=pltpu.CompilerParams(
            dimension_semantics=("arbitrary", "arbitrary")),
    )(mv, mc, gv, gam, we)


def _tc_h(mc, neg, alog, gneg, glog, gam, we):
    return pl.pallas_call(
        _tc_h_body,
        grid=(NC, VGRID),
        in_specs=[_edge_spec(), _edge_spec(), _edge_spec(), _edge_spec(),
                  _edge_spec(), _scal_spec(), _scal_spec()],
        out_specs=[_edge_spec(), _edge_spec()],
        out_shape=[jax.ShapeDtypeStruct((NC, RV, 128), jnp.float32)] * 2,
        compiler_params=pltpu.CompilerParams(
            dimension_semantics=("arbitrary", "arbitrary")),
    )(mc, neg, alog, gneg, glog, gam, we)


def _params_body(chn_ref, w1_ref, b1_ref, w2_ref, b2_ref, out_ref):
    x = chn_ref[...]
    ss = jnp.sum(x * x, axis=0, keepdims=True)
    s4 = ss[:, 0:32] + ss[:, 32:64] + ss[:, 64:96] + ss[:, 96:128]
    estat = s4 / float(N)
    inner = estat / (1.0 + jnp.sqrt(1.0 + estat)) / (4.0 * RATE)
    snr = 10.0 * jnp.log(inner) / 2.302585092994046

    rows = []
    for k in range(3):
        w1c = w1_ref[:, k:k + 1]
        b1c = b1_ref[:, k:k + 1]
        w2c = w2_ref[:, k:k + 1]
        h = jnp.clip(w1c * snr + b1c, 0.0, None)
        z = jnp.sum(h * w2c, axis=0, keepdims=True) + b2_ref[0, k]
        o = 1.0 / (1.0 + jnp.exp(-z))
        rows.append(jnp.concatenate(
            [jnp.concatenate([o[:, 0:16]] * 8, axis=1),
             jnp.concatenate([o[:, 16:32]] * 8, axis=1)], axis=0))
    out_ref[...] = jnp.stack(rows, axis=0)


def _tc_params(chn_v, w1t, b1t, w2t, b2t):
    return pl.pallas_call(
        _params_body,
        out_shape=jax.ShapeDtypeStruct((3, NC, 128), jnp.float32),
    )(chn_v, w1t, b1t, w2t, b2t)


NV = N // 4
WROWS = N_SP * L // 128


def _wiell_body(chn_ref, wi_ref, out_ref):
    out_ref[...] = wi_ref[...] * chn_ref[...]


def _tc_wiell(chn2p, wi):
    return pl.pallas_call(
        _wiell_body,
        grid=(NC,),
        in_specs=[pl.BlockSpec((1, WROWS, 128), lambda c: (c, 0, 0)),
                  pl.BlockSpec((1, 1, 128), lambda c: (c, 0, 0))],
        out_specs=pl.BlockSpec((1, WROWS, 128), lambda c: (c, 0, 0)),
        out_shape=jax.ShapeDtypeStruct((NC, WROWS, 128), jnp.float32),
        compiler_params=pltpu.CompilerParams(
            dimension_semantics=("arbitrary",)),
    )(chn2p, wi)


def kernel(chn_llr, edge_var, edge_chk, net_w1, net_b1, net_w2, net_b2):
    padv = jnp.full((E_PAD - E,), N, jnp.int32)
    padc = jnp.full((E_PAD - E,), M, jnp.int32)
    ev3d = jnp.concatenate([edge_var, padv]).reshape(NS, CH_PER_TILE, 128)
    ec3d = jnp.concatenate([edge_chk, padc]).reshape(NS, CH_PER_TILE, 128)
    zeros_slab = jnp.zeros((128, L), jnp.float32)

    w1t = net_w1.reshape(3, 20).T
    b1t = net_b1.T
    w2t = net_w2.reshape(3, 20).T
    b2t = net_b2.T
    chn_v = chn_llr.reshape(NV, 128)

    params = _tc_params(chn_v, w1t, b1t, w2t, b2t)
    gam = params[0].reshape(NC, 1, 128)
    wi = params[1].reshape(NC, 1, 128)
    we = params[2].reshape(NC, 1, 128)

    chn2 = chn_llr.reshape(N, NC, L).transpose(1, 0, 2)
    chn2p = jnp.concatenate(
        [chn2, jnp.zeros((NC, N_SP - N, L), jnp.float32)], axis=1)
    wiell = _tc_wiell(chn2p.reshape(NC, WROWS, 128), wi)
    wiell_sc = wiell.reshape(NC, N_SP, L)

    mv = jnp.zeros((NC, RV, 128), jnp.float32)
    mc = jnp.zeros((NC, RV, 128), jnp.float32)
    gv = _sc_gather_init(ev3d, wiell_sc).reshape(NC, RV, 128)
    sums_list = []
    for _ in range(T):
        mv, neg, alog = _tc_v(mv, mc, gv, gam, we)
        gneg, glog = _sc_chk(neg.reshape(NC, E_PAD, L),
                             alog.reshape(NC, E_PAD, L),
                             ec3d, zeros_slab)
        mc, a2 = _tc_h(mc, neg, alog,
                       gneg.reshape(NC, RV, 128), glog.reshape(NC, RV, 128),
                       gam, we)
        sums, gath = _sc_var(a2.reshape(NC, E_PAD, L), ev3d, wiell_sc)
        gv = gath.reshape(NC, RV, 128)
        sums_list.append(sums)

    out = jnp.stack(sums_list, axis=0)
    return out.transpose(0, 2, 1, 3).reshape(T, N, B)


# speedup vs baseline: 6.7183x; 1.2798x over previous
import functools
import jax
import jax.numpy as jnp
from jax import lax
from jax.experimental import pallas as pl
from jax.experimental.pallas import tpu as pltpu
from jax.experimental.pallas import tpu_sc as plsc

N = 50000
M = 25000
E = 200000
B = 32
T = 10
LLR_CLIP = 15.0
ABS_MIN = 1.1920930376163597e-06
RATE = 0.5

NC = 2
NS = 16
L = 16

E_PAD = 200704
CH_PER_TILE = 98
CH_PER_SLAB = 7
SLABS = 14
SLAB_ROWS = CH_PER_SLAB * 128
N_SP = 50048
M_SP = 25088
NROWS_TILE = N_SP // NS
MROWS_TILE = M_SP // NS

_mesh = plsc.VectorSubcoreMesh(core_axis_name="c", subcore_axis_name="s",
                               num_cores=NC, num_subcores=NS)


def _zero_rows(zeros_hbm, acc, base, nrows):
    pltpu.sync_copy(zeros_hbm.at[pl.ds(0, nrows)], acc.at[pl.ds(base, nrows)])


def _preload_acc(wiell, acc, c, s):
    r0 = s * NROWS_TILE
    pltpu.sync_copy(wiell.at[c, pl.ds(r0, NROWS_TILE)],
                    acc.at[pl.ds(r0, NROWS_TILE)])


def _gather_from_acc(acc, gath, idxbuf, gb0, gb1, gsem, wsem, c, tbase):
    def outer(g, _):
        s0 = 2 * g
        s1 = 2 * g + 1
        ws = []
        for j in range(CH_PER_SLAB):
            ws.append(pltpu.async_copy(
                acc.at[idxbuf.at[s0 * CH_PER_SLAB + j]],
                gb0.at[pl.ds(j * 128, 128)], gsem))
        for j in range(CH_PER_SLAB):
            ws.append(pltpu.async_copy(
                acc.at[idxbuf.at[s1 * CH_PER_SLAB + j]],
                gb1.at[pl.ds(j * 128, 128)], gsem))
        for w in ws:
            w.wait()
        w0 = pltpu.async_copy(
            gb0, gath.at[c, pl.ds(tbase + s0 * SLAB_ROWS, SLAB_ROWS)], wsem)
        w1 = pltpu.async_copy(
            gb1, gath.at[c, pl.ds(tbase + s1 * SLAB_ROWS, SLAB_ROWS)], wsem)
        w0.wait()
        w1.wait()
        return 0

    lax.fori_loop(0, SLABS // 2, outer, 0)


def _scatter_add_to_acc(vals, acc, idxbuf, vb0, vb1, lsem, ssem, c, tbase):
    def outer(g, _):
        s0 = 2 * g
        s1 = 2 * g + 1
        d0 = pltpu.async_copy(
            vals.at[c, pl.ds(tbase + s0 * SLAB_ROWS, SLAB_ROWS)], vb0, lsem)
        d1 = pltpu.async_copy(
            vals.at[c, pl.ds(tbase + s1 * SLAB_ROWS, SLAB_ROWS)], vb1, lsem)
        ws = []
        d0.wait()
        for j in range(CH_PER_SLAB):
            ws.append(pltpu.async_copy(
                vb0.at[pl.ds(j * 128, 128)],
                acc.at[idxbuf.at[s0 * CH_PER_SLAB + j]], ssem, add=True))
        d1.wait()
        for j in range(CH_PER_SLAB):
            ws.append(pltpu.async_copy(
                vb1.at[pl.ds(j * 128, 128)],
                acc.at[idxbuf.at[s1 * CH_PER_SLAB + j]], ssem, add=True))
        for w in ws:
            w.wait()
        return 0

    lax.fori_loop(0, SLABS // 2, outer, 0)


@functools.partial(
    pl.kernel,
    out_type=[jax.ShapeDtypeStruct((NC, N, L), jnp.float32),
              jax.ShapeDtypeStruct((NC, E_PAD, L), jnp.float32)],
    mesh=_mesh,
    scratch_types=[pltpu.VMEM((CH_PER_TILE, 128), jnp.int32),
                   pltpu.VMEM((SLAB_ROWS, L), jnp.float32),
                   pltpu.VMEM((SLAB_ROWS, L), jnp.float32),
                   pltpu.VMEM((SLAB_ROWS, L), jnp.float32),
                   pltpu.VMEM((SLAB_ROWS, L), jnp.float32),
                   pltpu.VMEM_SHARED((N_SP, L), jnp.float32),
                   pltpu.SemaphoreType.DMA,
                   pltpu.SemaphoreType.DMA,
                   pltpu.SemaphoreType.DMA,
                   pltpu.SemaphoreType.DMA],
    compiler_params=pltpu.CompilerParams(use_tc_tiling_on_sc=False),
)
def _sc_var(vals, idx3d, wiell, sums, gath,
            idxbuf, vb0, vb1, gb0, gb1, acc, lsem, ssem, gsem, wsem):
    c = lax.axis_index("c")
    s = lax.axis_index("s")
    tbase = s * (CH_PER_TILE * 128)

    pltpu.sync_copy(idx3d.at[s], idxbuf)
    _preload_acc(wiell, acc, c, s)
    plsc.subcore_barrier()

    _scatter_add_to_acc(vals, acc, idxbuf, vb0, vb1, lsem, ssem, c, tbase)
    plsc.subcore_barrier()

    r0 = s * NROWS_TILE
    pltpu.sync_copy(acc.at[pl.ds(r0, 3080)], sums.at[c, pl.ds(r0, 3080)])
    t0 = jnp.minimum(r0 + 3080, N - 48)
    pltpu.sync_copy(acc.at[pl.ds(t0, 48)], sums.at[c, pl.ds(t0, 48)])

    _gather_from_acc(acc, gath, idxbuf, gb0, gb1, gsem, wsem, c, tbase)


@functools.partial(
    pl.kernel,
    out_type=jax.ShapeDtypeStruct((NC, E_PAD, L), jnp.float32),
    mesh=_mesh,
    scratch_types=[pltpu.VMEM((CH_PER_TILE, 128), jnp.int32),
                   pltpu.VMEM((SLAB_ROWS, L), jnp.float32),
                   pltpu.VMEM((SLAB_ROWS, L), jnp.float32),
                   pltpu.VMEM_SHARED((N_SP, L), jnp.float32),
                   pltpu.SemaphoreType.DMA,
                   pltpu.SemaphoreType.DMA],
    compiler_params=pltpu.CompilerParams(use_tc_tiling_on_sc=False),
)
def _sc_gather_init(idx3d, wiell, gath, idxbuf, gb0, gb1, acc, gsem, wsem):
    c = lax.axis_index("c")
    s = lax.axis_index("s")
    tbase = s * (CH_PER_TILE * 128)

    pltpu.sync_copy(idx3d.at[s], idxbuf)
    _preload_acc(wiell, acc, c, s)
    plsc.subcore_barrier()
    _gather_from_acc(acc, gath, idxbuf, gb0, gb1, gsem, wsem, c, tbase)


@functools.partial(
    pl.kernel,
    out_type=[jax.ShapeDtypeStruct((NC, E_PAD, L), jnp.float32),
              jax.ShapeDtypeStruct((NC, E_PAD, L), jnp.float32)],
    mesh=_mesh,
    scratch_types=[pltpu.VMEM((CH_PER_TILE, 128), jnp.int32),
                   pltpu.VMEM((SLAB_ROWS, L), jnp.float32),
                   pltpu.VMEM((SLAB_ROWS, L), jnp.float32),
                   pltpu.VMEM((SLAB_ROWS, L), jnp.float32),
                   pltpu.VMEM((SLAB_ROWS, L), jnp.float32),
                   pltpu.VMEM_SHARED((M_SP, L), jnp.float32),
                   pltpu.VMEM_SHARED((M_SP, L), jnp.float32),
                   pltpu.SemaphoreType.DMA,
                   pltpu.SemaphoreType.DMA,
                   pltpu.SemaphoreType.DMA,
                   pltpu.SemaphoreType.DMA],
    compiler_params=pltpu.CompilerParams(use_tc_tiling_on_sc=False),
)
def _sc_chk(valsA, valsB, idx3d, zeros_hbm, gathA, gathB,
            idxbuf, vb0, vb1, gb0, gb1, accA, accB, lsem, ssem, gsem, wsem):
    c = lax.axis_index("c")
    s = lax.axis_index("s")
    tbase = s * (CH_PER_TILE * 128)

    pltpu.sync_copy(idx3d.at[s], idxbuf)
    _zero_rows(zeros_hbm, accA, s * MROWS_TILE, MROWS_TILE)
    _zero_rows(zeros_hbm, accB, s * MROWS_TILE, MROWS_TILE)
    plsc.subcore_barrier()

    _scatter_add_to_acc(valsA, accA, idxbuf, vb0, vb1, lsem, ssem, c, tbase)
    _scatter_add_to_acc(valsB, accB, idxbuf, vb0, vb1, lsem, ssem, c, tbase)
    plsc.subcore_barrier()

    _gather_from_acc(accA, gathA, idxbuf, gb0, gb1, gsem, wsem, c, tbase)
    _gather_from_acc(accB, gathB, idxbuf, gb0, gb1, gsem, wsem, c, tbase)


RV = E_PAD * L // 128
VBLK = 512
VGRID = RV // VBLK


def _edge_spec():
    return pl.BlockSpec((1, VBLK, 128), lambda c, i: (c, i, 0))


def _scal_spec():
    return pl.BlockSpec((1, 1, 128), lambda c, i: (c, 0, 0))


def _tc_v_body(mv_ref, mc_ref, gv_ref, g_ref, we_ref,
               mvo_ref, neg_ref, alog_ref):
    g = g_ref[...]
    we = we_ref[...]
    mv = (1.0 - g) * mv_ref[...] + g * (gv_ref[...] - we * mc_ref[...])
    mvo_ref[...] = mv
    lam = jnp.clip(mv, -LLR_CLIP, LLR_CLIP)
    neg_ref[...] = jnp.where(lam < 0, 1.0, 0.0).astype(jnp.float32)
    ab = jnp.clip(jnp.abs(lam), ABS_MIN, LLR_CLIP)
    alog_ref[...] = jnp.log(jnp.tanh(ab * 0.5))


def _tc_h_body(mc_ref, neg_ref, alog_ref, gneg_ref, glog_ref, g_ref, we_ref,
               mco_ref, a2_ref):
    g = g_ref[...]
    we = we_ref[...]
    neg = neg_ref[...]
    parity = jnp.mod(gneg_ref[...] - neg, 2.0)
    sgn = 1.0 - 2.0 * parity
    amp = glog_ref[...] - alog_ref[...]
    t = jnp.exp(amp) * (1.0 - 1e-6)
    h = sgn * jnp.log((1.0 + t) / (1.0 - t))
    mc = (1.0 - g) * mc_ref[...] + g * h
    mco_ref[...] = mc
    a2_ref[...] = we * mc


def _tc_v(mv, mc, gv, gam, we):
    return pl.pallas_call(
        _tc_v_body,
        grid=(NC, VGRID),
        in_specs=[_edge_spec(), _edge_spec(), _edge_spec(),
                  _scal_spec(), _scal_spec()],
        out_specs=[_edge_spec(), _edge_spec(), _edge_spec()],
        out_shape=[jax.ShapeDtypeStruct((NC, RV, 128), jnp.float32)] * 3,
        compiler_params=pltpu.CompilerParams(
            dimension_semantics=("arbitrary", "arbitrary")),
    )(mv, mc, gv, gam, we)


def _tc_h(mc, neg, alog, gneg, glog, gam, we):
    return pl.pallas_call(
        _tc_h_body,
        grid=(NC, VGRID),
        in_specs=[_edge_spec(), _edge_spec(), _edge_spec(), _edge_spec(),
                  _edge_spec(), _scal_spec(), _scal_spec()],
        out_specs=[_edge_spec(), _edge_spec()],
        out_shape=[jax.ShapeDtypeStruct((NC, RV, 128), jnp.float32)] * 2,
        compiler_params=pltpu.CompilerParams(
            dimension_semantics=("arbitrary", "arbitrary")),
    )(mc, neg, alog, gneg, glog, gam, we)


def _adaptive_params(chn_llr, net_w1, net_b1, net_w2, net_b2):
    Estat = jnp.mean(chn_llr ** 2, axis=0)
    snr_hat = 10.0 * jnp.log10(Estat / (1.0 + jnp.sqrt(1.0 + Estat))
                               / (4.0 * RATE))
    x = snr_hat.reshape((-1, 1))
    h = jnp.clip(jnp.einsum('bi,khi->bkh', x, net_w1) + net_b1[None, :, :],
                 0.0, None)
    o = jax.nn.sigmoid(jnp.einsum('bkh,koh->bko', h, net_w2)
                       + net_b2[None, :, :])
    return o.squeeze(-1).T


NV = N // 4
WROWS = N_SP * L // 128


def _wiell_body(chn_ref, wi_ref, out_ref):
    out_ref[...] = wi_ref[...] * chn_ref[...]


def _tc_wiell(chn2p, wi):
    return pl.pallas_call(
        _wiell_body,
        grid=(NC,),
        in_specs=[pl.BlockSpec((1, WROWS, 128), lambda c: (c, 0, 0)),
                  pl.BlockSpec((1, 1, 128), lambda c: (c, 0, 0))],
        out_specs=pl.BlockSpec((1, WROWS, 128), lambda c: (c, 0, 0)),
        out_shape=jax.ShapeDtypeStruct((NC, WROWS, 128), jnp.float32),
        compiler_params=pltpu.CompilerParams(
            dimension_semantics=("arbitrary",)),
    )(chn2p, wi)


def kernel(chn_llr, edge_var, edge_chk, net_w1, net_b1, net_w2, net_b2):
    padv = jnp.full((E_PAD - E,), N, jnp.int32)
    padc = jnp.full((E_PAD - E,), M, jnp.int32)
    ev3d = jnp.concatenate([edge_var, padv]).reshape(NS, CH_PER_TILE, 128)
    ec3d = jnp.concatenate([edge_chk, padc]).reshape(NS, CH_PER_TILE, 128)
    zeros_slab = jnp.zeros((MROWS_TILE, L), jnp.float32)

    params = _adaptive_params(chn_llr, net_w1, net_b1, net_w2, net_b2)
    gam = jnp.tile(params[0].reshape(NC, 1, L), (1, 1, 8))
    wi = jnp.tile(params[1].reshape(NC, 1, L), (1, 1, 8))
    we = jnp.tile(params[2].reshape(NC, 1, L), (1, 1, 8))

    chn2 = chn_llr.reshape(N, NC, L).transpose(1, 0, 2)
    chn2p = jnp.concatenate(
        [chn2, jnp.zeros((NC, N_SP - N, L), jnp.float32)], axis=1)
    wiell = _tc_wiell(chn2p.reshape(NC, WROWS, 128), wi)
    wiell_sc = wiell.reshape(NC, N_SP, L)

    mv = jnp.zeros((NC, RV, 128), jnp.float32)
    mc = jnp.zeros((NC, RV, 128), jnp.float32)
    gv = _sc_gather_init(ev3d, wiell_sc).reshape(NC, RV, 128)
    sums_list = []
    for _ in range(T):
        mv, neg, alog = _tc_v(mv, mc, gv, gam, we)
        gneg, glog = _sc_chk(neg.reshape(NC, E_PAD, L),
                             alog.reshape(NC, E_PAD, L),
                             ec3d, zeros_slab)
        mc, a2 = _tc_h(mc, neg, alog,
                       gneg.reshape(NC, RV, 128), glog.reshape(NC, RV, 128),
                       gam, we)
        sums, gath = _sc_var(a2.reshape(NC, E_PAD, L), ev3d, wiell_sc)
        gv = gath.reshape(NC, RV, 128)
        sums_list.append(sums)

    out = jnp.stack(sums_list, axis=0)
    return out.transpose(0, 2, 1, 3).reshape(T, N, B)


# speedup vs baseline: 7.0168x; 1.0444x over previous
import functools
import jax
import jax.numpy as jnp
from jax import lax
from jax.experimental import pallas as pl
from jax.experimental.pallas import tpu as pltpu
from jax.experimental.pallas import tpu_sc as plsc

N = 50000
M = 25000
E = 200000
B = 32
T = 10
LLR_CLIP = 15.0
ABS_MIN = 1.1920930376163597e-06
RATE = 0.5

NC = 2
NS = 16
L = 16

E_PAD = 200704
CH_PER_TILE = 98
CH_PER_SLAB = 7
SLABS = 14
SLAB_ROWS = CH_PER_SLAB * 128
N_SP = 50048
M_SP = 25088
NROWS_TILE = N_SP // NS
MROWS_TILE = M_SP // NS

_mesh = plsc.VectorSubcoreMesh(core_axis_name="c", subcore_axis_name="s",
                               num_cores=NC, num_subcores=NS)


def _zero_rows(zeros_hbm, acc, base, nrows):
    pltpu.sync_copy(zeros_hbm.at[pl.ds(0, nrows)], acc.at[pl.ds(base, nrows)])


def _preload_acc(wiell, acc, c, s):
    r0 = s * NROWS_TILE
    pltpu.sync_copy(wiell.at[c, pl.ds(r0, NROWS_TILE)],
                    acc.at[pl.ds(r0, NROWS_TILE)])


def _gather_from_acc(acc, gath, idxbuf, b0, b1, b2, b3, gsem, wsem, c, tbase):
    def fire(slab, buf):
        ws = []
        for j in range(CH_PER_SLAB):
            ws.append(pltpu.async_copy(
                acc.at[idxbuf.at[slab * CH_PER_SLAB + j]],
                buf.at[pl.ds(j * 128, 128)], gsem))
        return ws

    def wb(slab, buf):
        return pltpu.async_copy(
            buf, gath.at[c, pl.ds(tbase + slab * SLAB_ROWS, SLAB_ROWS)], wsem)

    def body(g, _):
        s = 4 * g
        wsA = fire(s, b0) + fire(s + 1, b1)
        for w in wsA:
            w.wait()
        wA0 = wb(s, b0)
        wA1 = wb(s + 1, b1)
        wsB = fire(s + 2, b2) + fire(s + 3, b3)
        wA0.wait()
        wA1.wait()
        for w in wsB:
            w.wait()
        wB0 = wb(s + 2, b2)
        wB1 = wb(s + 3, b3)
        wB0.wait()
        wB1.wait()
        return 0

    lax.fori_loop(0, 3, body, 0)
    ws = fire(12, b0) + fire(13, b1)
    for w in ws:
        w.wait()
    w0 = wb(12, b0)
    w1 = wb(13, b1)
    w0.wait()
    w1.wait()


def _scatter_add_to_acc(vals, acc, idxbuf, b0, b1, b2, b3, lsem, ssem,
                        c, tbase):
    def load(slab, buf):
        return pltpu.async_copy(
            vals.at[c, pl.ds(tbase + slab * SLAB_ROWS, SLAB_ROWS)], buf, lsem)

    def fire(slab, buf):
        ws = []
        for j in range(CH_PER_SLAB):
            ws.append(pltpu.async_copy(
                buf.at[pl.ds(j * 128, 128)],
                acc.at[idxbuf.at[slab * CH_PER_SLAB + j]], ssem, add=True))
        return ws

    l0 = load(0, b0)
    l1 = load(1, b1)
    l2 = load(2, b2)
    l3 = load(3, b3)

    def body(g, _):
        s = 4 * g
        ldrainA0 = pltpu.make_async_copy(
            vals.at[c, pl.ds(tbase, SLAB_ROWS)], b0, lsem)
        ldrainA0.wait()
        ldrainA1 = pltpu.make_async_copy(
            vals.at[c, pl.ds(tbase, SLAB_ROWS)], b1, lsem)
        ldrainA1.wait()
        wsA = fire(s, b0) + fire(s + 1, b1)
        pltpu.make_async_copy(vals.at[c, pl.ds(tbase, SLAB_ROWS)], b2, lsem).wait()
        pltpu.make_async_copy(vals.at[c, pl.ds(tbase, SLAB_ROWS)], b3, lsem).wait()
        for w in wsA:
            w.wait()

        load(s + 4, b0)
        load(s + 5, b1)

        wsB = fire(s + 2, b2) + fire(s + 3, b3)
        for w in wsB:
            w.wait()

        @pl.when(g < 2)
        def _():
            load(s + 6, b2)
            load(s + 7, b3)

        return 0

    lax.fori_loop(0, 3, body, 0)
    pltpu.make_async_copy(vals.at[c, pl.ds(tbase, SLAB_ROWS)], b0, lsem).wait()
    pltpu.make_async_copy(vals.at[c, pl.ds(tbase, SLAB_ROWS)], b1, lsem).wait()
    ws = fire(12, b0) + fire(13, b1)
    for w in ws:
        w.wait()


@functools.partial(
    pl.kernel,
    out_type=[jax.ShapeDtypeStruct((NC, N, L), jnp.float32),
              jax.ShapeDtypeStruct((NC, E_PAD, L), jnp.float32)],
    mesh=_mesh,
    scratch_types=[pltpu.VMEM((CH_PER_TILE, 128), jnp.int32),
                   pltpu.VMEM((SLAB_ROWS, L), jnp.float32),
                   pltpu.VMEM((SLAB_ROWS, L), jnp.float32),
                   pltpu.VMEM((SLAB_ROWS, L), jnp.float32),
                   pltpu.VMEM((SLAB_ROWS, L), jnp.float32),
                   pltpu.VMEM_SHARED((N_SP, L), jnp.float32),
                   pltpu.SemaphoreType.DMA,
                   pltpu.SemaphoreType.DMA,
                   pltpu.SemaphoreType.DMA,
                   pltpu.SemaphoreType.DMA],
    compiler_params=pltpu.CompilerParams(use_tc_tiling_on_sc=False),
)
def _sc_var(vals, idx3d, wiell, sums, gath,
            idxbuf, vb0, vb1, gb0, gb1, acc, lsem, ssem, gsem, wsem):
    c = lax.axis_index("c")
    s = lax.axis_index("s")
    tbase = s * (CH_PER_TILE * 128)

    pltpu.sync_copy(idx3d.at[s], idxbuf)
    _preload_acc(wiell, acc, c, s)
    plsc.subcore_barrier()

    _scatter_add_to_acc(vals, acc, idxbuf, vb0, vb1, gb0, gb1, lsem, ssem,
                        c, tbase)
    plsc.subcore_barrier()

    r0 = s * NROWS_TILE
    pltpu.sync_copy(acc.at[pl.ds(r0, 3080)], sums.at[c, pl.ds(r0, 3080)])
    t0 = jnp.minimum(r0 + 3080, N - 48)
    pltpu.sync_copy(acc.at[pl.ds(t0, 48)], sums.at[c, pl.ds(t0, 48)])

    _gather_from_acc(acc, gath, idxbuf, vb0, vb1, gb0, gb1, gsem, wsem,
                     c, tbase)


@functools.partial(
    pl.kernel,
    out_type=jax.ShapeDtypeStruct((NC, E_PAD, L), jnp.float32),
    mesh=_mesh,
    scratch_types=[pltpu.VMEM((CH_PER_TILE, 128), jnp.int32),
                   pltpu.VMEM((SLAB_ROWS, L), jnp.float32),
                   pltpu.VMEM((SLAB_ROWS, L), jnp.float32),
                   pltpu.VMEM((SLAB_ROWS, L), jnp.float32),
                   pltpu.VMEM((SLAB_ROWS, L), jnp.float32),
                   pltpu.VMEM_SHARED((N_SP, L), jnp.float32),
                   pltpu.SemaphoreType.DMA,
                   pltpu.SemaphoreType.DMA],
    compiler_params=pltpu.CompilerParams(use_tc_tiling_on_sc=False),
)
def _sc_gather_init(idx3d, wiell, gath, idxbuf, b0, b1, b2, b3, acc,
                    gsem, wsem):
    c = lax.axis_index("c")
    s = lax.axis_index("s")
    tbase = s * (CH_PER_TILE * 128)

    pltpu.sync_copy(idx3d.at[s], idxbuf)
    _preload_acc(wiell, acc, c, s)
    plsc.subcore_barrier()
    _gather_from_acc(acc, gath, idxbuf, b0, b1, b2, b3, gsem, wsem,
                     c, tbase)


@functools.partial(
    pl.kernel,
    out_type=[jax.ShapeDtypeStruct((NC, E_PAD, L), jnp.float32),
              jax.ShapeDtypeStruct((NC, E_PAD, L), jnp.float32)],
    mesh=_mesh,
    scratch_types=[pltpu.VMEM((CH_PER_TILE, 128), jnp.int32),
                   pltpu.VMEM((SLAB_ROWS, L), jnp.float32),
                   pltpu.VMEM((SLAB_ROWS, L), jnp.float32),
                   pltpu.VMEM((SLAB_ROWS, L), jnp.float32),
                   pltpu.VMEM((SLAB_ROWS, L), jnp.float32),
                   pltpu.VMEM_SHARED((M_SP, L), jnp.float32),
                   pltpu.VMEM_SHARED((M_SP, L), jnp.float32),
                   pltpu.SemaphoreType.DMA,
                   pltpu.SemaphoreType.DMA,
                   pltpu.SemaphoreType.DMA,
                   pltpu.SemaphoreType.DMA],
    compiler_params=pltpu.CompilerParams(use_tc_tiling_on_sc=False),
)
def _sc_chk(valsA, valsB, idx3d, zeros_hbm, gathA, gathB,
            idxbuf, vb0, vb1, gb0, gb1, accA, accB, lsem, ssem, gsem, wsem):
    c = lax.axis_index("c")
    s = lax.axis_index("s")
    tbase = s * (CH_PER_TILE * 128)

    pltpu.sync_copy(idx3d.at[s], idxbuf)
    _zero_rows(zeros_hbm, accA, s * MROWS_TILE, MROWS_TILE)
    _zero_rows(zeros_hbm, accB, s * MROWS_TILE, MROWS_TILE)
    plsc.subcore_barrier()

    _scatter_add_to_acc(valsA, accA, idxbuf, vb0, vb1, gb0, gb1, lsem, ssem,
                        c, tbase)
    _scatter_add_to_acc(valsB, accB, idxbuf, vb0, vb1, gb0, gb1, lsem, ssem,
                        c, tbase)
    plsc.subcore_barrier()

    _gather_from_acc(accA, gathA, idxbuf, vb0, vb1, gb0, gb1, gsem, wsem,
                     c, tbase)
    _gather_from_acc(accB, gathB, idxbuf, vb0, vb1, gb0, gb1, gsem, wsem,
                     c, tbase)


RV = E_PAD * L // 128
VBLK = 512
VGRID = RV // VBLK


def _edge_spec():
    return pl.BlockSpec((1, VBLK, 128), lambda c, i: (c, i, 0))


def _scal_spec():
    return pl.BlockSpec((1, 1, 128), lambda c, i: (c, 0, 0))


def _tc_v_body(mv_ref, mc_ref, gv_ref, g_ref, we_ref,
               mvo_ref, neg_ref, alog_ref):
    g = g_ref[...]
    we = we_ref[...]
    mv = (1.0 - g) * mv_ref[...] + g * (gv_ref[...] - we * mc_ref[...])
    mvo_ref[...] = mv
    lam = jnp.clip(mv, -LLR_CLIP, LLR_CLIP)
    neg_ref[...] = jnp.where(lam < 0, 1.0, 0.0).astype(jnp.float32)
    ab = jnp.clip(jnp.abs(lam), ABS_MIN, LLR_CLIP)
    alog_ref[...] = jnp.log(jnp.tanh(ab * 0.5))


def _tc_h_body(mc_ref, neg_ref, alog_ref, gneg_ref, glog_ref, g_ref, we_ref,
               mco_ref, a2_ref):
    g = g_ref[...]
    we = we_ref[...]
    neg = neg_ref[...]
    parity = jnp.mod(gneg_ref[...] - neg, 2.0)
    sgn = 1.0 - 2.0 * parity
    amp = glog_ref[...] - alog_ref[...]
    t = jnp.exp(amp) * (1.0 - 1e-6)
    h = sgn * jnp.log((1.0 + t) / (1.0 - t))
    mc = (1.0 - g) * mc_ref[...] + g * h
    mco_ref[...] = mc
    a2_ref[...] = we * mc


def _tc_v(mv, mc, gv, gam, we):
    return pl.pallas_call(
        _tc_v_body,
        grid=(NC, VGRID),
        in_specs=[_edge_spec(), _edge_spec(), _edge_spec(),
                  _scal_spec(), _scal_spec()],
        out_specs=[_edge_spec(), _edge_spec(), _edge_spec()],
        out_shape=[jax.ShapeDtypeStruct((NC, RV, 128), jnp.float32)] * 3,
        compiler_params=pltpu.CompilerParams(
            dimension_semantics=("arbitrary", "arbitrary")),
    )(mv, mc, gv, gam, we)


def _tc_h(mc, neg, alog, gneg, glog, gam, we):
    return pl.pallas_call(
        _tc_h_body,
        grid=(NC, VGRID),
        in_specs=[_edge_spec(), _edge_spec(), _edge_spec(), _edge_spec(),
                  _edge_spec(), _scal_spec(), _scal_spec()],
        out_specs=[_edge_spec(), _edge_spec()],
        out_shape=[jax.ShapeDtypeStruct((NC, RV, 128), jnp.float32)] * 2,
        compiler_params=pltpu.CompilerParams(
            dimension_semantics=("arbitrary", "arbitrary")),
    )(mc, neg, alog, gneg, glog, gam, we)


def _adaptive_params(chn_llr, net_w1, net_b1, net_w2, net_b2):
    Estat = jnp.mean(chn_llr ** 2, axis=0)
    snr_hat = 10.0 * jnp.log10(Estat / (1.0 + jnp.sqrt(1.0 + Estat))
                               / (4.0 * RATE))
    x = snr_hat.reshape((-1, 1))
    h = jnp.clip(jnp.einsum('bi,khi->bkh', x, net_w1) + net_b1[None, :, :],
                 0.0, None)
    o = jax.nn.sigmoid(jnp.einsum('bkh,koh->bko', h, net_w2)
                       + net_b2[None, :, :])
    return o.squeeze(-1).T


NV = N // 4
WROWS = N_SP * L // 128


def _wiell_body(chn_ref, wi_ref, out_ref):
    out_ref[...] = wi_ref[...] * chn_ref[...]


def _tc_wiell(chn2p, wi):
    return pl.pallas_call(
        _wiell_body,
        grid=(NC,),
        in_specs=[pl.BlockSpec((1, WROWS, 128), lambda c: (c, 0, 0)),
                  pl.BlockSpec((1, 1, 128), lambda c: (c, 0, 0))],
        out_specs=pl.BlockSpec((1, WROWS, 128), lambda c: (c, 0, 0)),
        out_shape=jax.ShapeDtypeStruct((NC, WROWS, 128), jnp.float32),
        compiler_params=pltpu.CompilerParams(
            dimension_semantics=("arbitrary",)),
    )(chn2p, wi)


def kernel(chn_llr, edge_var, edge_chk, net_w1, net_b1, net_w2, net_b2):
    padv = jnp.full((E_PAD - E,), N, jnp.int32)
    padc = jnp.full((E_PAD - E,), M, jnp.int32)
    ev3d = jnp.concatenate([edge_var, padv]).reshape(NS, CH_PER_TILE, 128)
    ec3d = jnp.concatenate([edge_chk, padc]).reshape(NS, CH_PER_TILE, 128)
    zeros_slab = jnp.zeros((MROWS_TILE, L), jnp.float32)

    params = _adaptive_params(chn_llr, net_w1, net_b1, net_w2, net_b2)
    gam = jnp.tile(params[0].reshape(NC, 1, L), (1, 1, 8))
    wi = jnp.tile(params[1].reshape(NC, 1, L), (1, 1, 8))
    we = jnp.tile(params[2].reshape(NC, 1, L), (1, 1, 8))

    chn2 = chn_llr.reshape(N, NC, L).transpose(1, 0, 2)
    chn2p = jnp.concatenate(
        [chn2, jnp.zeros((NC, N_SP - N, L), jnp.float32)], axis=1)
    wiell = _tc_wiell(chn2p.reshape(NC, WROWS, 128), wi)
    wiell_sc = wiell.reshape(NC, N_SP, L)

    mv = jnp.zeros((NC, RV, 128), jnp.float32)
    mc = jnp.zeros((NC, RV, 128), jnp.float32)
    gv = _sc_gather_init(ev3d, wiell_sc).reshape(NC, RV, 128)
    sums_list = []
    for _ in range(T):
        mv, neg, alog = _tc_v(mv, mc, gv, gam, we)
        gneg, glog = _sc_chk(neg.reshape(NC, E_PAD, L),
                             alog.reshape(NC, E_PAD, L),
                             ec3d, zeros_slab)
        mc, a2 = _tc_h(mc, neg, alog,
                       gneg.reshape(NC, RV, 128), glog.reshape(NC, RV, 128),
                       gam, we)
        sums, gath = _sc_var(a2.reshape(NC, E_PAD, L), ev3d, wiell_sc)
        gv = gath.reshape(NC, RV, 128)
        sums_list.append(sums)

    out = jnp.stack(sums_list, axis=0)
    return out.transpose(0, 2, 1, 3).reshape(T, N, B)


# speedup vs baseline: 7.1633x; 1.0209x over previous
import functools
import jax
import jax.numpy as jnp
from jax import lax
from jax.experimental import pallas as pl
from jax.experimental.pallas import tpu as pltpu
from jax.experimental.pallas import tpu_sc as plsc

N = 50000
M = 25000
E = 200000
B = 32
T = 10
LLR_CLIP = 15.0
ABS_MIN = 1.1920930376163597e-06
RATE = 0.5

NC = 2
NS = 16
L = 16

E_PAD = 200704
CH_PER_TILE = 98
CH_PER_SLAB = 7
SLABS = 14
SLAB_ROWS = CH_PER_SLAB * 128
N_SP = 50048
M_SP = 25088
NROWS_TILE = N_SP // NS
MROWS_TILE = M_SP // NS

_mesh = plsc.VectorSubcoreMesh(core_axis_name="c", subcore_axis_name="s",
                               num_cores=NC, num_subcores=NS)


def _zero_rows(zeros_hbm, acc, base, nrows):
    pltpu.sync_copy(zeros_hbm.at[pl.ds(0, nrows)], acc.at[pl.ds(base, nrows)])


def _preload_acc(wiell, acc, c, s):
    r0 = s * NROWS_TILE
    pltpu.sync_copy(wiell.at[c, pl.ds(r0, NROWS_TILE)],
                    acc.at[pl.ds(r0, NROWS_TILE)])


NBUF = 4


def _scatter_pipe(jobs, idxbuf, bufs, lsem, ssem, c, cb):
    n = len(jobs)

    def load(i):
        v, _, sl = jobs[i]
        return pltpu.async_copy(v.at[c, pl.ds(cb + sl * SLAB_ROWS,
                                              SLAB_ROWS)],
                                bufs[i % NBUF], lsem)

    def stream(i):
        _, a, sl = jobs[i]
        buf = bufs[i % NBUF]
        return [pltpu.async_copy(buf.at[pl.ds(j * 128, 128)],
                                 a.at[idxbuf.at[sl * CH_PER_SLAB + j]],
                                 ssem, add=True)
                for j in range(CH_PER_SLAB)]

    ld = {}
    st = {}
    for i in range(min(2, n)):
        ld[i] = load(i)
    for i in range(n):
        if i - 2 >= 0:
            for w in st[i - 2]:
                w.wait()
        if i + 2 < n:
            ld[i + 2] = load(i + 2)
        ld[i].wait()
        st[i] = stream(i)
    for w in st[n - 2] + st[n - 1]:
        w.wait()


def _gather_pipe(jobs, idxbuf, bufs, gsem, wsem, c, cb):
    n = len(jobs)

    def gat(i):
        _, a, sl = jobs[i]
        buf = bufs[i % NBUF]
        return [pltpu.async_copy(a.at[idxbuf.at[sl * CH_PER_SLAB + j]],
                                 buf.at[pl.ds(j * 128, 128)], gsem)
                for j in range(CH_PER_SLAB)]

    def wb(i):
        g, _, sl = jobs[i]
        return pltpu.async_copy(bufs[i % NBUF],
                                g.at[c, pl.ds(cb + sl * SLAB_ROWS,
                                              SLAB_ROWS)], wsem)

    gd = {}
    wd = {}
    for i in range(min(2, n)):
        gd[i] = gat(i)
    for i in range(n):
        if i - 2 >= 0:
            wd[i - 2].wait()
        if i + 2 < n:
            gd[i + 2] = gat(i + 2)
        for w in gd[i]:
            w.wait()
        wd[i] = wb(i)
    wd[n - 2].wait()
    wd[n - 1].wait()


_SC_BUFS = [pltpu.VMEM((SLAB_ROWS, L), jnp.float32)] * NBUF


@functools.partial(
    pl.kernel,
    out_type=[jax.ShapeDtypeStruct((NC, N, L), jnp.float32),
              jax.ShapeDtypeStruct((NC, E_PAD, L), jnp.float32)],
    mesh=_mesh,
    scratch_types=[pltpu.VMEM((CH_PER_TILE, 128), jnp.int32)] + _SC_BUFS +
                  [pltpu.VMEM_SHARED((N_SP, L), jnp.float32),
                   pltpu.SemaphoreType.DMA,
                   pltpu.SemaphoreType.DMA,
                   pltpu.SemaphoreType.DMA,
                   pltpu.SemaphoreType.DMA],
    compiler_params=pltpu.CompilerParams(use_tc_tiling_on_sc=False),
)
def _sc_var(vals, idx3d, wiell, sums, gath,
            idxbuf, b0, b1, b2, b3, acc, lsem, ssem, gsem, wsem):
    c = lax.axis_index("c")
    s = lax.axis_index("s")
    cb = s * (CH_PER_TILE * 128)
    bufs = [b0, b1, b2, b3]

    pltpu.sync_copy(idx3d.at[s], idxbuf)
    _preload_acc(wiell, acc, c, s)
    plsc.subcore_barrier()

    _scatter_pipe([(vals, acc, sl) for sl in range(SLABS)],
                  idxbuf, bufs, lsem, ssem, c, cb)
    plsc.subcore_barrier()

    r0 = s * NROWS_TILE
    for k in range(3):
        pltpu.sync_copy(acc.at[pl.ds(r0 + k * SLAB_ROWS, SLAB_ROWS)], b0)
        pltpu.sync_copy(b0, sums.at[c, pl.ds(r0 + k * SLAB_ROWS, SLAB_ROWS)])
    t0 = jnp.minimum(r0 + 3 * SLAB_ROWS, N - 440)
    pltpu.sync_copy(acc.at[pl.ds(t0, 440)], b0.at[pl.ds(0, 440)])
    pltpu.sync_copy(b0.at[pl.ds(0, 440)], sums.at[c, pl.ds(t0, 440)])

    _gather_pipe([(gath, acc, sl) for sl in range(SLABS)],
                 idxbuf, bufs, gsem, wsem, c, cb)


@functools.partial(
    pl.kernel,
    out_type=[jax.ShapeDtypeStruct((NC, E_PAD, L), jnp.float32),
              jax.ShapeDtypeStruct((NC, E_PAD, L), jnp.float32)],
    mesh=_mesh,
    scratch_types=[pltpu.VMEM((CH_PER_TILE, 128), jnp.int32)] + _SC_BUFS +
                  [pltpu.VMEM_SHARED((M_SP, L), jnp.float32),
                   pltpu.VMEM_SHARED((M_SP, L), jnp.float32),
                   pltpu.SemaphoreType.DMA,
                   pltpu.SemaphoreType.DMA,
                   pltpu.SemaphoreType.DMA,
                   pltpu.SemaphoreType.DMA],
    compiler_params=pltpu.CompilerParams(use_tc_tiling_on_sc=False),
)
def _sc_chk(valsA, valsB, idx3d, zeros_hbm, gathA, gathB,
            idxbuf, b0, b1, b2, b3, accA, accB,
            lsem, ssem, gsem, wsem):
    c = lax.axis_index("c")
    s = lax.axis_index("s")
    cb = s * (CH_PER_TILE * 128)
    bufs = [b0, b1, b2, b3]

    pltpu.sync_copy(idx3d.at[s], idxbuf)
    _zero_rows(zeros_hbm, accA, s * MROWS_TILE, MROWS_TILE)
    _zero_rows(zeros_hbm, accB, s * MROWS_TILE, MROWS_TILE)
    plsc.subcore_barrier()

    sjobs = []
    gjobs = []
    for sl in range(SLABS):
        sjobs += [(valsA, accA, sl), (valsB, accB, sl)]
        gjobs += [(gathA, accA, sl), (gathB, accB, sl)]
    _scatter_pipe(sjobs, idxbuf, bufs, lsem, ssem, c, cb)
    plsc.subcore_barrier()
    _gather_pipe(gjobs, idxbuf, bufs, gsem, wsem, c, cb)


RV = E_PAD * L // 128
VBLK = 512
VGRID = RV // VBLK


def _edge_spec():
    return pl.BlockSpec((1, VBLK, 128), lambda c, i: (c, i, 0))


def _scal_spec():
    return pl.BlockSpec((1, 1, 128), lambda c, i: (c, 0, 0))


def _tc_v_body(mv_ref, mc_ref, gv_ref, g_ref, we_ref,
               mvo_ref, neg_ref, alog_ref):
    g = g_ref[...]
    we = we_ref[...]
    mv = (1.0 - g) * mv_ref[...] + g * (gv_ref[...] - we * mc_ref[...])
    mvo_ref[...] = mv
    lam = jnp.clip(mv, -LLR_CLIP, LLR_CLIP)
    neg_ref[...] = jnp.where(lam < 0, 1.0, 0.0).astype(jnp.float32)
    ab = jnp.clip(jnp.abs(lam), ABS_MIN, LLR_CLIP)
    alog_ref[...] = jnp.log(jnp.tanh(ab * 0.5))


def _tc_h_body(mc_ref, neg_ref, alog_ref, gneg_ref, glog_ref, g_ref, we_ref,
               mco_ref, a2_ref):
    g = g_ref[...]
    we = we_ref[...]
    neg = neg_ref[...]
    parity = jnp.mod(gneg_ref[...] - neg, 2.0)
    sgn = 1.0 - 2.0 * parity
    amp = glog_ref[...] - alog_ref[...]
    t = jnp.exp(amp) * (1.0 - 1e-6)
    h = sgn * jnp.log((1.0 + t) / (1.0 - t))
    mc = (1.0 - g) * mc_ref[...] + g * h
    mco_ref[...] = mc
    a2_ref[...] = we * mc


def _tc_v(mv, mc, gv, gam, we):
    return pl.pallas_call(
        _tc_v_body,
        grid=(NC, VGRID),
        in_specs=[_edge_spec(), _edge_spec(), _edge_spec(),
                  _scal_spec(), _scal_spec()],
        out_specs=[_edge_spec(), _edge_spec(), _edge_spec()],
        out_shape=[jax.ShapeDtypeStruct((NC, RV, 128), jnp.float32)] * 3,
        compiler_params=pltpu.CompilerParams(
            dimension_semantics=("arbitrary", "arbitrary")),
    )(mv, mc, gv, gam, we)


def _tc_h(mc, neg, alog, gneg, glog, gam, we):
    return pl.pallas_call(
        _tc_h_body,
        grid=(NC, VGRID),
        in_specs=[_edge_spec(), _edge_spec(), _edge_spec(), _edge_spec(),
                  _edge_spec(), _scal_spec(), _scal_spec()],
        out_specs=[_edge_spec(), _edge_spec()],
        out_shape=[jax.ShapeDtypeStruct((NC, RV, 128), jnp.float32)] * 2,
        compiler_params=pltpu.CompilerParams(
            dimension_semantics=("arbitrary", "arbitrary")),
    )(mc, neg, alog, gneg, glog, gam, we)


def _adaptive_params(chn_llr, net_w1, net_b1, net_w2, net_b2):
    Estat = jnp.mean(chn_llr ** 2, axis=0)
    snr_hat = 10.0 * jnp.log10(Estat / (1.0 + jnp.sqrt(1.0 + Estat))
                               / (4.0 * RATE))
    x = snr_hat.reshape((-1, 1))
    h = jnp.clip(jnp.einsum('bi,khi->bkh', x, net_w1) + net_b1[None, :, :],
                 0.0, None)
    o = jax.nn.sigmoid(jnp.einsum('bkh,koh->bko', h, net_w2)
                       + net_b2[None, :, :])
    return o.squeeze(-1).T


NV = N // 4
WROWS = N_SP * L // 128


def _wiell_body(chn_ref, wi_ref, out_ref):
    out_ref[...] = wi_ref[...] * chn_ref[...]


def _tc_wiell(chn2p, wi):
    return pl.pallas_call(
        _wiell_body,
        grid=(NC,),
        in_specs=[pl.BlockSpec((1, WROWS, 128), lambda c: (c, 0, 0)),
                  pl.BlockSpec((1, 1, 128), lambda c: (c, 0, 0))],
        out_specs=pl.BlockSpec((1, WROWS, 128), lambda c: (c, 0, 0)),
        out_shape=jax.ShapeDtypeStruct((NC, WROWS, 128), jnp.float32),
        compiler_params=pltpu.CompilerParams(
            dimension_semantics=("arbitrary",)),
    )(chn2p, wi)


def kernel(chn_llr, edge_var, edge_chk, net_w1, net_b1, net_w2, net_b2):
    padv = jnp.full((E_PAD - E,), N, jnp.int32)
    padc = jnp.full((E_PAD - E,), M, jnp.int32)
    ev3d = jnp.concatenate([edge_var, padv]).reshape(NS, CH_PER_TILE, 128)
    ec3d = jnp.concatenate([edge_chk, padc]).reshape(NS, CH_PER_TILE, 128)
    zeros_slab = jnp.zeros((MROWS_TILE, L), jnp.float32)

    params = _adaptive_params(chn_llr, net_w1, net_b1, net_w2, net_b2)
    gam = jnp.tile(params[0].reshape(NC, 1, L), (1, 1, 8))
    wi = jnp.tile(params[1].reshape(NC, 1, L), (1, 1, 8))
    we = jnp.tile(params[2].reshape(NC, 1, L), (1, 1, 8))

    chn2 = chn_llr.reshape(N, NC, L).transpose(1, 0, 2)
    chn2p = jnp.concatenate(
        [chn2, jnp.zeros((NC, N_SP - N, L), jnp.float32)], axis=1)
    wiell = _tc_wiell(chn2p.reshape(NC, WROWS, 128), wi)
    wiell_sc = wiell.reshape(NC, N_SP, L)

    mv = jnp.zeros((NC, RV, 128), jnp.float32)
    mc = jnp.zeros((NC, RV, 128), jnp.float32)
    zero_e = jnp.zeros((NC, E_PAD, L), jnp.float32)
    gv = _sc_var(zero_e, ev3d, wiell_sc)[1].reshape(NC, RV, 128)
    sums_list = []
    for _ in range(T):
        mv, neg, alog = _tc_v(mv, mc, gv, gam, we)
        gneg, glog = _sc_chk(neg.reshape(NC, E_PAD, L),
                             alog.reshape(NC, E_PAD, L),
                             ec3d, zeros_slab)
        mc, a2 = _tc_h(mc, neg, alog,
                       gneg.reshape(NC, RV, 128), glog.reshape(NC, RV, 128),
                       gam, we)
        sums, gath = _sc_var(a2.reshape(NC, E_PAD, L),
                             ev3d, wiell_sc)
        gv = gath.reshape(NC, RV, 128)
        sums_list.append(sums)

    out = jnp.stack(sums_list, axis=0)
    return out.transpose(0, 2, 1, 3).reshape(T, N, B)


# speedup vs baseline: 7.2708x; 1.0150x over previous
import functools
import jax
import jax.numpy as jnp
from jax import lax
from jax.experimental import pallas as pl
from jax.experimental.pallas import tpu as pltpu
from jax.experimental.pallas import tpu_sc as plsc

N = 50000
M = 25000
E = 200000
B = 32
T = 10
LLR_CLIP = 15.0
ABS_MIN = 1.1920930376163597e-06
RATE = 0.5

NC = 2
NS = 16
L = 16

E_PAD = 200704
CH_PER_TILE = 98
CH_PER_SLAB = 7
SLABS = 14
SLAB_ROWS = CH_PER_SLAB * 128
N_SP = 50048
M_SP = 25088
NROWS_TILE = N_SP // NS
MROWS_TILE = M_SP // NS

_mesh = plsc.VectorSubcoreMesh(core_axis_name="c", subcore_axis_name="s",
                               num_cores=NC, num_subcores=NS)


def _zero_rows(zeros_hbm, acc, base, nrows):
    pltpu.sync_copy(zeros_hbm.at[pl.ds(0, nrows)], acc.at[pl.ds(base, nrows)])


def _preload_acc(wiell, acc, c, s):
    r0 = s * NROWS_TILE
    pltpu.sync_copy(wiell.at[c, pl.ds(r0, NROWS_TILE)],
                    acc.at[pl.ds(r0, NROWS_TILE)])


NBUF = 4


def _scatter_pipe(jobs, idxbuf, bufs, lsem, ssem, c, cb):
    n = len(jobs)

    def load(i):
        v, _, sl = jobs[i]
        return pltpu.async_copy(v.at[c, pl.ds(cb + sl * SLAB_ROWS,
                                              SLAB_ROWS)],
                                bufs[i % NBUF], lsem)

    def stream(i):
        _, a, sl = jobs[i]
        buf = bufs[i % NBUF]
        return [pltpu.async_copy(buf.at[pl.ds(j * 128, 128)],
                                 a.at[idxbuf.at[sl * CH_PER_SLAB + j]],
                                 ssem, add=True)
                for j in range(CH_PER_SLAB)]

    ld = {}
    st = {}
    for i in range(min(2, n)):
        ld[i] = load(i)
    for i in range(n):
        if i - 2 >= 0:
            for w in st[i - 2]:
                w.wait()
        if i + 2 < n:
            ld[i + 2] = load(i + 2)
        ld[i].wait()
        st[i] = stream(i)
    for w in st[n - 2] + st[n - 1]:
        w.wait()


def _gather_pipe(jobs, idxbuf, bufs, gsem, wsem, c, cb):
    n = len(jobs)

    def gat(i):
        _, a, sl = jobs[i]
        buf = bufs[i % NBUF]
        return [pltpu.async_copy(a.at[idxbuf.at[sl * CH_PER_SLAB + j]],
                                 buf.at[pl.ds(j * 128, 128)], gsem)
                for j in range(CH_PER_SLAB)]

    def wb(i):
        g, _, sl = jobs[i]
        return pltpu.async_copy(bufs[i % NBUF],
                                g.at[c, pl.ds(cb + sl * SLAB_ROWS,
                                              SLAB_ROWS)], wsem)

    gd = {}
    wd = {}
    for i in range(min(2, n)):
        gd[i] = gat(i)
    for i in range(n):
        if i - 2 >= 0:
            wd[i - 2].wait()
        if i + 2 < n:
            gd[i + 2] = gat(i + 2)
        for w in gd[i]:
            w.wait()
        wd[i] = wb(i)
    wd[n - 2].wait()
    wd[n - 1].wait()


_SC_BUFS = [pltpu.VMEM((SLAB_ROWS, L), jnp.float32)] * NBUF


@functools.partial(
    pl.kernel,
    out_type=[jax.ShapeDtypeStruct((NC, N, L), jnp.float32),
              jax.ShapeDtypeStruct((NC, E_PAD, L), jnp.float32)],
    mesh=_mesh,
    scratch_types=[pltpu.VMEM((CH_PER_TILE, 128), jnp.int32)] + _SC_BUFS +
                  [pltpu.VMEM_SHARED((N_SP, L), jnp.float32),
                   pltpu.SemaphoreType.DMA,
                   pltpu.SemaphoreType.DMA,
                   pltpu.SemaphoreType.DMA,
                   pltpu.SemaphoreType.DMA],
    compiler_params=pltpu.CompilerParams(use_tc_tiling_on_sc=False),
)
def _sc_var(vals, idx3d, wiell, sums, gath,
            idxbuf, b0, b1, b2, b3, acc, lsem, ssem, gsem, wsem):
    c = lax.axis_index("c")
    s = lax.axis_index("s")
    cb = s * (CH_PER_TILE * 128)
    bufs = [b0, b1, b2, b3]

    pltpu.sync_copy(idx3d.at[s], idxbuf)
    _preload_acc(wiell, acc, c, s)
    plsc.subcore_barrier()

    _scatter_pipe([(vals, acc, sl) for sl in range(SLABS)],
                  idxbuf, bufs, lsem, ssem, c, cb)
    plsc.subcore_barrier()

    r0 = s * NROWS_TILE
    for k in range(3):
        pltpu.sync_copy(acc.at[pl.ds(r0 + k * SLAB_ROWS, SLAB_ROWS)], b0)
        pltpu.sync_copy(b0, sums.at[c, pl.ds(r0 + k * SLAB_ROWS, SLAB_ROWS)])
    t0 = jnp.minimum(r0 + 3 * SLAB_ROWS, N - 440)
    pltpu.sync_copy(acc.at[pl.ds(t0, 440)], b0.at[pl.ds(0, 440)])
    pltpu.sync_copy(b0.at[pl.ds(0, 440)], sums.at[c, pl.ds(t0, 440)])

    _gather_pipe([(gath, acc, sl) for sl in range(SLABS)],
                 idxbuf, bufs, gsem, wsem, c, cb)


@functools.partial(
    pl.kernel,
    out_type=jax.ShapeDtypeStruct((NC, E_PAD, L), jnp.float32),
    mesh=_mesh,
    scratch_types=[pltpu.VMEM((CH_PER_TILE, 128), jnp.int32)] + _SC_BUFS +
                  [pltpu.VMEM_SHARED((N_SP, L), jnp.float32),
                   pltpu.SemaphoreType.DMA,
                   pltpu.SemaphoreType.DMA],
    compiler_params=pltpu.CompilerParams(use_tc_tiling_on_sc=False),
)
def _sc_gather_init(idx3d, wiell, gath, idxbuf, b0, b1, b2, b3, acc,
                    gsem, wsem):
    c = lax.axis_index("c")
    s = lax.axis_index("s")
    cb = s * (CH_PER_TILE * 128)
    bufs = [b0, b1, b2, b3]

    pltpu.sync_copy(idx3d.at[s], idxbuf)
    _preload_acc(wiell, acc, c, s)
    plsc.subcore_barrier()
    _gather_pipe([(gath, acc, sl) for sl in range(SLABS)],
                 idxbuf, bufs, gsem, wsem, c, cb)


@functools.partial(
    pl.kernel,
    out_type=jax.ShapeDtypeStruct((NC, N, L), jnp.float32),
    mesh=_mesh,
    scratch_types=[pltpu.VMEM((CH_PER_TILE, 128), jnp.int32)] + _SC_BUFS +
                  [pltpu.VMEM_SHARED((N_SP, L), jnp.float32),
                   pltpu.SemaphoreType.DMA,
                   pltpu.SemaphoreType.DMA],
    compiler_params=pltpu.CompilerParams(use_tc_tiling_on_sc=False),
)
def _sc_var_last(vals, idx3d, wiell, sums, idxbuf, b0, b1, b2, b3, acc,
                 lsem, ssem):
    c = lax.axis_index("c")
    s = lax.axis_index("s")
    cb = s * (CH_PER_TILE * 128)
    bufs = [b0, b1, b2, b3]

    pltpu.sync_copy(idx3d.at[s], idxbuf)
    _preload_acc(wiell, acc, c, s)
    plsc.subcore_barrier()
    _scatter_pipe([(vals, acc, sl) for sl in range(SLABS)],
                  idxbuf, bufs, lsem, ssem, c, cb)
    plsc.subcore_barrier()
    r0 = s * NROWS_TILE
    for k in range(3):
        pltpu.sync_copy(acc.at[pl.ds(r0 + k * SLAB_ROWS, SLAB_ROWS)], b0)
        pltpu.sync_copy(b0, sums.at[c, pl.ds(r0 + k * SLAB_ROWS, SLAB_ROWS)])
    t0 = jnp.minimum(r0 + 3 * SLAB_ROWS, N - 440)
    pltpu.sync_copy(acc.at[pl.ds(t0, 440)], b0.at[pl.ds(0, 440)])
    pltpu.sync_copy(b0.at[pl.ds(0, 440)], sums.at[c, pl.ds(t0, 440)])


@functools.partial(
    pl.kernel,
    out_type=[jax.ShapeDtypeStruct((NC, E_PAD, L), jnp.float32),
              jax.ShapeDtypeStruct((NC, E_PAD, L), jnp.float32)],
    mesh=_mesh,
    scratch_types=[pltpu.VMEM((CH_PER_TILE, 128), jnp.int32)] + _SC_BUFS +
                  [pltpu.VMEM_SHARED((M_SP, L), jnp.float32),
                   pltpu.VMEM_SHARED((M_SP, L), jnp.float32),
                   pltpu.SemaphoreType.DMA,
                   pltpu.SemaphoreType.DMA,
                   pltpu.SemaphoreType.DMA,
                   pltpu.SemaphoreType.DMA],
    compiler_params=pltpu.CompilerParams(use_tc_tiling_on_sc=False),
)
def _sc_chk(valsA, valsB, idx3d, zeros_hbm, gathA, gathB,
            idxbuf, b0, b1, b2, b3, accA, accB,
            lsem, ssem, gsem, wsem):
    c = lax.axis_index("c")
    s = lax.axis_index("s")
    cb = s * (CH_PER_TILE * 128)
    bufs = [b0, b1, b2, b3]

    pltpu.sync_copy(idx3d.at[s], idxbuf)
    _zero_rows(zeros_hbm, accA, s * MROWS_TILE, MROWS_TILE)
    _zero_rows(zeros_hbm, accB, s * MROWS_TILE, MROWS_TILE)
    plsc.subcore_barrier()

    sjobs = []
    gjobs = []
    for sl in range(SLABS):
        sjobs += [(valsA, accA, sl), (valsB, accB, sl)]
        gjobs += [(gathA, accA, sl), (gathB, accB, sl)]
    _scatter_pipe(sjobs, idxbuf, bufs, lsem, ssem, c, cb)
    plsc.subcore_barrier()
    _gather_pipe(gjobs, idxbuf, bufs, gsem, wsem, c, cb)


RV = E_PAD * L // 128
VBLK = 512
VGRID = RV // VBLK


def _edge_spec():
    return pl.BlockSpec((1, VBLK, 128), lambda c, i: (c, i, 0))


def _scal_spec():
    return pl.BlockSpec((1, 1, 128), lambda c, i: (c, 0, 0))


def _tc_v_body(mv_ref, mc_ref, gv_ref, g_ref, we_ref,
               mvo_ref, neg_ref, alog_ref):
    g = g_ref[...]
    we = we_ref[...]
    mv = (1.0 - g) * mv_ref[...] + g * (gv_ref[...] - we * mc_ref[...])
    mvo_ref[...] = mv
    lam = jnp.clip(mv, -LLR_CLIP, LLR_CLIP)
    neg_ref[...] = jnp.where(lam < 0, 1.0, 0.0).astype(jnp.float32)
    ab = jnp.clip(jnp.abs(lam), ABS_MIN, LLR_CLIP)
    alog_ref[...] = jnp.log(jnp.tanh(ab * 0.5))


def _tc_h_body(mc_ref, neg_ref, alog_ref, gneg_ref, glog_ref, g_ref, we_ref,
               mco_ref, a2_ref):
    g = g_ref[...]
    we = we_ref[...]
    neg = neg_ref[...]
    parity = jnp.mod(gneg_ref[...] - neg, 2.0)
    sgn = 1.0 - 2.0 * parity
    amp = glog_ref[...] - alog_ref[...]
    t = jnp.exp(amp) * (1.0 - 1e-6)
    h = sgn * jnp.log((1.0 + t) / (1.0 - t))
    mc = (1.0 - g) * mc_ref[...] + g * h
    mco_ref[...] = mc
    a2_ref[...] = we * mc


def _tc_v(mv, mc, gv, gam, we):
    return pl.pallas_call(
        _tc_v_body,
        grid=(NC, VGRID),
        in_specs=[_edge_spec(), _edge_spec(), _edge_spec(),
                  _scal_spec(), _scal_spec()],
        out_specs=[_edge_spec(), _edge_spec(), _edge_spec()],
        out_shape=[jax.ShapeDtypeStruct((NC, RV, 128), jnp.float32)] * 3,
        compiler_params=pltpu.CompilerParams(
            dimension_semantics=("arbitrary", "arbitrary")),
    )(mv, mc, gv, gam, we)


def _tc_h(mc, neg, alog, gneg, glog, gam, we):
    return pl.pallas_call(
        _tc_h_body,
        grid=(NC, VGRID),
        in_specs=[_edge_spec(), _edge_spec(), _edge_spec(), _edge_spec(),
                  _edge_spec(), _scal_spec(), _scal_spec()],
        out_specs=[_edge_spec(), _edge_spec()],
        out_shape=[jax.ShapeDtypeStruct((NC, RV, 128), jnp.float32)] * 2,
        compiler_params=pltpu.CompilerParams(
            dimension_semantics=("arbitrary", "arbitrary")),
    )(mc, neg, alog, gneg, glog, gam, we)


def _adaptive_params(chn_llr, net_w1, net_b1, net_w2, net_b2):
    Estat = jnp.mean(chn_llr ** 2, axis=0)
    snr_hat = 10.0 * jnp.log10(Estat / (1.0 + jnp.sqrt(1.0 + Estat))
                               / (4.0 * RATE))
    x = snr_hat.reshape((-1, 1))
    h = jnp.clip(jnp.einsum('bi,khi->bkh', x, net_w1) + net_b1[None, :, :],
                 0.0, None)
    o = jax.nn.sigmoid(jnp.einsum('bkh,koh->bko', h, net_w2)
                       + net_b2[None, :, :])
    return o.squeeze(-1).T


NV = N // 4
WROWS = N_SP * L // 128


def _wiell_body(chn_ref, wi_ref, out_ref):
    out_ref[...] = wi_ref[...] * chn_ref[...]


def _tc_wiell(chn2p, wi):
    return pl.pallas_call(
        _wiell_body,
        grid=(NC,),
        in_specs=[pl.BlockSpec((1, WROWS, 128), lambda c: (c, 0, 0)),
                  pl.BlockSpec((1, 1, 128), lambda c: (c, 0, 0))],
        out_specs=pl.BlockSpec((1, WROWS, 128), lambda c: (c, 0, 0)),
        out_shape=jax.ShapeDtypeStruct((NC, WROWS, 128), jnp.float32),
        compiler_params=pltpu.CompilerParams(
            dimension_semantics=("arbitrary",)),
    )(chn2p, wi)


def kernel(chn_llr, edge_var, edge_chk, net_w1, net_b1, net_w2, net_b2):
    padv = jnp.full((E_PAD - E,), N, jnp.int32)
    padc = jnp.full((E_PAD - E,), M, jnp.int32)
    ev3d = jnp.concatenate([edge_var, padv]).reshape(NS, CH_PER_TILE, 128)
    ec3d = jnp.concatenate([edge_chk, padc]).reshape(NS, CH_PER_TILE, 128)
    zeros_slab = jnp.zeros((MROWS_TILE, L), jnp.float32)

    params = _adaptive_params(chn_llr, net_w1, net_b1, net_w2, net_b2)
    gam = jnp.tile(params[0].reshape(NC, 1, L), (1, 1, 8))
    wi = jnp.tile(params[1].reshape(NC, 1, L), (1, 1, 8))
    we = jnp.tile(params[2].reshape(NC, 1, L), (1, 1, 8))

    chn2 = chn_llr.reshape(N, NC, L).transpose(1, 0, 2)
    chn2p = jnp.concatenate(
        [chn2, jnp.zeros((NC, N_SP - N, L), jnp.float32)], axis=1)
    wiell = _tc_wiell(chn2p.reshape(NC, WROWS, 128), wi)
    wiell_sc = wiell.reshape(NC, N_SP, L)

    mv = jnp.zeros((NC, RV, 128), jnp.float32)
    mc = jnp.zeros((NC, RV, 128), jnp.float32)
    gv = _sc_gather_init(ev3d, wiell_sc).reshape(NC, RV, 128)
    sums_list = []
    for t in range(T):
        mv, neg, alog = _tc_v(mv, mc, gv, gam, we)
        gneg, glog = _sc_chk(neg.reshape(NC, E_PAD, L),
                             alog.reshape(NC, E_PAD, L),
                             ec3d, zeros_slab)
        mc, a2 = _tc_h(mc, neg, alog,
                       gneg.reshape(NC, RV, 128), glog.reshape(NC, RV, 128),
                       gam, we)
        if t < T - 1:
            sums, gath = _sc_var(a2.reshape(NC, E_PAD, L), ev3d, wiell_sc)
            gv = gath.reshape(NC, RV, 128)
        else:
            sums = _sc_var_last(a2.reshape(NC, E_PAD, L), ev3d, wiell_sc)
        sums_list.append(sums)

    out = jnp.stack(sums_list, axis=0)
    return out.transpose(0, 2, 1, 3).reshape(T, N, B)
